# baseline (speedup 1.0000x reference)
import functools

import jax
import jax.numpy as jnp
from jax import lax
from jax.experimental import pallas as pl
from jax.experimental.pallas import tpu as pltpu
from jax.experimental.pallas import tpu_sc as plsc

_BATCH = 4
_SEQ = 8192
_DIM = 1024
_NUM_WORKERS = 32
_ROWS_PER_W = _SEQ // _NUM_WORKERS
_CHUNK = 32
_NBUF = 3


_TC_BLOCK = 1024
_SC_BATCH = 1
_TC_BATCH = _BATCH - _SC_BATCH


def _tc_broadcast(table):
    def body(t_ref, o_ref):
        t = t_ref[...]
        for b in range(_TC_BATCH):
            o_ref[b] = t

    return pl.pallas_call(
        body,
        grid=(_SEQ // _TC_BLOCK,),
        in_specs=[pl.BlockSpec((_TC_BLOCK, _DIM), lambda i: (i, 0))],
        out_specs=pl.BlockSpec((_TC_BATCH, _TC_BLOCK, _DIM), lambda i: (0, i, 0)),
        out_shape=jax.ShapeDtypeStruct((_TC_BATCH, _SEQ, _DIM), jnp.float32),
    )(table)


_CHUNK_SC = 64


def _sc_part(table):
    mesh = plsc.VectorSubcoreMesh(core_axis_name="c", subcore_axis_name="s")

    @functools.partial(
        pl.kernel,
        mesh=mesh,
        out_type=jax.ShapeDtypeStruct((_SC_BATCH, _SEQ, _DIM), jnp.float32),
        scratch_types=[pltpu.VMEM((_CHUNK_SC, _DIM), jnp.float32)],
    )
    def k(table_hbm, out_hbm, buf):
        wid = lax.axis_index("s") * 2 + lax.axis_index("c")
        base = wid * _ROWS_PER_W
        for i in range(_ROWS_PER_W // _CHUNK_SC):
            row = base + i * _CHUNK_SC
            pltpu.sync_copy(table_hbm.at[pl.ds(row, _CHUNK_SC)], buf)
            for b in range(_SC_BATCH):
                pltpu.sync_copy(buf, out_hbm.at[b, pl.ds(row, _CHUNK_SC)])

    return k(table)


def _sc_broadcast(table):
    tc_out = _tc_broadcast(table)
    sc_out = _sc_part(table)
    return jnp.concatenate([tc_out, sc_out], axis=0)


def kernel(x, table):
    del x
    return _sc_broadcast(table)


# speedup vs baseline: 2.2116x; 2.2116x over previous
import functools

import jax
import jax.numpy as jnp
from jax import lax
from jax.experimental import pallas as pl
from jax.experimental.pallas import tpu as pltpu
from jax.experimental.pallas import tpu_sc as plsc

_BATCH = 4
_SEQ = 8192
_DIM = 1024
_NUM_WORKERS = 32
_ROWS_PER_W = _SEQ // _NUM_WORKERS
_CHUNK = 64


def _sc_broadcast(table):
    mesh = plsc.VectorSubcoreMesh(core_axis_name="c", subcore_axis_name="s")

    @functools.partial(
        pl.kernel,
        mesh=mesh,
        out_type=jax.ShapeDtypeStruct((_BATCH, _SEQ, _DIM), jnp.float32),
        scratch_types=[
            pltpu.VMEM((_CHUNK, _DIM), jnp.float32),
            pltpu.SemaphoreType.DMA,
        ],
    )
    def k(table_hbm, out_hbm, buf, wsem):
        wid = lax.axis_index("s") * 2 + lax.axis_index("c")
        base = wid * _ROWS_PER_W
        for i in range(_ROWS_PER_W // _CHUNK):
            row = base + i * _CHUNK
            pltpu.sync_copy(table_hbm.at[pl.ds(row, _CHUNK)], buf)
            writes = [
                pltpu.async_copy(buf, out_hbm.at[b, pl.ds(row, _CHUNK)], wsem)
                for b in range(_BATCH)
            ]
            for w in writes:
                w.wait()

    return k(table)


def kernel(x, table):
    del x
    return _sc_broadcast(table)


# speedup vs baseline: 2.2607x; 1.0222x over previous
import functools

import jax
import jax.numpy as jnp
from jax import lax
from jax.experimental import pallas as pl
from jax.experimental.pallas import tpu as pltpu
from jax.experimental.pallas import tpu_sc as plsc

_BATCH = 4
_SEQ = 8192
_DIM = 1024
_NUM_CORES = 2
_NUM_SUBCORES = 16
_NUM_WORKERS = _NUM_CORES * _NUM_SUBCORES
_ROWS_PER_W = _SEQ // _NUM_WORKERS
_CHUNK = 64
_NCHUNK = _ROWS_PER_W // _CHUNK


def _sc_broadcast(table):
    mesh = plsc.VectorSubcoreMesh(core_axis_name="c", subcore_axis_name="s")

    @functools.partial(
        pl.kernel,
        mesh=mesh,
        out_type=jax.ShapeDtypeStruct((_BATCH, _SEQ, _DIM), jnp.float32),
        scratch_types=[
            pltpu.VMEM((_CHUNK, _DIM), jnp.float32),
            pltpu.VMEM_SHARED((_NUM_SUBCORES, _CHUNK, _DIM), jnp.float32),
            pltpu.SemaphoreType.DMA,
            pltpu.SemaphoreType.DMA,
            pltpu.SemaphoreType.DMA,
            pltpu.SemaphoreType.DMA,
        ],
    )
    def k(table_hbm, out_hbm, buf0, shared, rsem0, rsem1, wsem0, wsem1):
        sid = lax.axis_index("s")
        wid = sid * _NUM_CORES + lax.axis_index("c")
        base = wid * _ROWS_PER_W
        bufs = (buf0, shared.at[sid])
        rsems = (rsem0, rsem1)
        wsems = (wsem0, wsem1)

        def issue_read(j):
            row = base + j * _CHUNK
            return pltpu.async_copy(
                table_hbm.at[pl.ds(row, _CHUNK)], bufs[j % 2], rsems[j % 2]
            )

        rd = [None] * _NCHUNK
        wr = [None] * _NCHUNK
        rd[0] = issue_read(0)
        rd[1] = issue_read(1)
        for i in range(_NCHUNK):
            b = i % 2
            row = base + i * _CHUNK
            rd[i].wait()
            wr[i] = [
                pltpu.async_copy(bufs[b], out_hbm.at[bb, pl.ds(row, _CHUNK)], wsems[b])
                for bb in range(_BATCH)
            ]
            nxt = i + 1
            if 2 <= nxt < _NCHUNK:
                for h in wr[nxt - 2]:
                    h.wait()
                rd[nxt] = issue_read(nxt)
        for i in (_NCHUNK - 2, _NCHUNK - 1):
            for h in wr[i]:
                h.wait()

    return k(table)


def kernel(x, table):
    del x
    return _sc_broadcast(table)
